# baseline (speedup 1.0000x reference)
import functools

import jax
import jax.numpy as jnp
from jax import lax
from jax.experimental import pallas as pl
from jax.experimental.pallas import tpu as pltpu
from jax.experimental.pallas import tpu_sc as plsc

DIM = 64
PAD = 0
LANES = 16

N_CORES = 2
N_SUBCORES = 16
NW = N_CORES * N_SUBCORES

CHUNK = 128


def _emb_body(idx_hbm, table_hbm, out_hbm, idx_v, rows_v, sem, *, n_chunks):
    cid = lax.axis_index("c")
    sid = lax.axis_index("s")
    wid = sid * N_CORES + cid
    base = wid * (n_chunks * CHUNK)

    pltpu.sync_copy(idx_hbm.at[wid], idx_v)

    zeros = jnp.zeros((LANES,), jnp.float32)

    def chunk_body(c, carry):
        pltpu.async_copy(table_hbm.at[idx_v.at[c]], rows_v, sem).wait()

        def group_body(g, carry2):
            iv = idx_v[c, pl.ds(g * LANES, LANES)]
            m = iv == PAD
            npad = plsc.all_reduce_population_count(m)[0]

            @pl.when(npad > 0)
            def _zero_pad_rows():
                rows16 = g * LANES + lax.iota(jnp.int32, LANES)
                for col in range(DIM):
                    plsc.store_scatter(
                        rows_v,
                        [rows16, jnp.full((LANES,), col, jnp.int32)],
                        zeros,
                        mask=m,
                    )

            return carry2

        lax.fori_loop(0, CHUNK // LANES, group_body, 0)

        pltpu.sync_copy(rows_v, out_hbm.at[pl.ds(base + c * CHUNK, CHUNK)])
        return carry

    lax.fori_loop(0, n_chunks, chunk_body, 0)


def kernel(src_input, emb_table):
    L, B, _ = src_input.shape
    total = L * B
    assert total % (NW * CHUNK) == 0
    n_chunks = total // (NW * CHUNK)

    idx = src_input.reshape(NW, n_chunks, CHUNK)

    mesh = plsc.VectorSubcoreMesh(core_axis_name="c", subcore_axis_name="s")
    run = functools.partial(
        pl.kernel,
        mesh=mesh,
        out_type=jax.ShapeDtypeStruct((total, DIM), jnp.float32),
        scratch_types=[
            pltpu.VMEM((n_chunks, CHUNK), jnp.int32),
            pltpu.VMEM((CHUNK, DIM), jnp.float32),
            pltpu.SemaphoreType.DMA,
        ],
        compiler_params=pltpu.CompilerParams(
            needs_layout_passes=False, use_tc_tiling_on_sc=False
        ),
    )(functools.partial(_emb_body, n_chunks=n_chunks))

    out = run(idx, emb_table)
    return out.reshape(L, B, DIM)


# speedup vs baseline: 1.0616x; 1.0616x over previous
import functools

import jax
import jax.numpy as jnp
from jax import lax
from jax.experimental import pallas as pl
from jax.experimental.pallas import tpu as pltpu
from jax.experimental.pallas import tpu_sc as plsc

DIM = 64
PAD = 0
LANES = 16

N_CORES = 2
N_SUBCORES = 16
NW = N_CORES * N_SUBCORES

CHUNK = 128
SUPER = 5
SROWS = SUPER * CHUNK


def _emb_body(idx_hbm, table_hbm, out_hbm, idx_v, rows_v, gsem, *, n_super):
    cid = lax.axis_index("c")
    sid = lax.axis_index("s")
    wid = sid * N_CORES + cid
    base = wid * (n_super * SROWS)

    pltpu.sync_copy(idx_hbm.at[wid], idx_v)

    zeros = jnp.zeros((LANES,), jnp.float32)

    def fire(sup, buf):
        for j in range(SUPER):
            pltpu.async_copy(
                table_hbm.at[idx_v.at[sup * SUPER + j]],
                rows_v.at[buf, pl.ds(j * CHUNK, CHUNK)],
                gsem,
            )

    def drain(sup, buf):
        for j in range(SUPER):
            pltpu.make_async_copy(
                table_hbm.at[idx_v.at[sup * SUPER + j]],
                rows_v.at[buf, pl.ds(j * CHUNK, CHUNK)],
                gsem,
            ).wait()

    def process(sup, buf):
        def group_body(g, carry):
            cidx = sup * SUPER + g // (CHUNK // LANES)
            off = (g % (CHUNK // LANES)) * LANES
            iv = idx_v[cidx, pl.ds(off, LANES)]
            m = iv == PAD
            npad = plsc.all_reduce_population_count(m)[0]

            @pl.when(npad > 0)
            def _zero_pad_rows():
                rows16 = g * LANES + lax.iota(jnp.int32, LANES)
                for col in range(DIM):
                    plsc.store_scatter(
                        rows_v.at[buf],
                        [rows16, jnp.full((LANES,), col, jnp.int32)],
                        zeros,
                        mask=m,
                    )

            return carry

        lax.fori_loop(0, SROWS // LANES, group_body, 0)

    def write(sup, buf):
        pltpu.sync_copy(
            rows_v.at[buf], out_hbm.at[pl.ds(base + sup * SROWS, SROWS)]
        )

    fire(0, 0)

    def pipe_body(t, carry):
        s0 = 2 * t
        s1 = 2 * t + 1

        drain(s0, 0)
        fire(s1, 1)
        process(s0, 0)
        write(s0, 0)

        drain(s1, 1)

        @pl.when(s1 + 1 < n_super)
        def _fire_next():
            fire(s1 + 1, 0)

        process(s1, 1)
        write(s1, 1)
        return carry

    lax.fori_loop(0, n_super // 2, pipe_body, 0)


def kernel(src_input, emb_table):
    L, B, _ = src_input.shape
    total = L * B
    assert total % (NW * SROWS) == 0 and (total // (NW * SROWS)) % 2 == 0
    n_super = total // (NW * SROWS)

    idx = src_input.reshape(NW, n_super * SUPER, CHUNK)

    mesh = plsc.VectorSubcoreMesh(core_axis_name="c", subcore_axis_name="s")
    run = functools.partial(
        pl.kernel,
        mesh=mesh,
        out_type=jax.ShapeDtypeStruct((total, DIM), jnp.float32),
        scratch_types=[
            pltpu.VMEM((n_super * SUPER, CHUNK), jnp.int32),
            pltpu.VMEM((2, SROWS, DIM), jnp.float32),
            pltpu.SemaphoreType.DMA,
        ],
        compiler_params=pltpu.CompilerParams(
            needs_layout_passes=False, use_tc_tiling_on_sc=False
        ),
    )(functools.partial(_emb_body, n_super=n_super))

    out = run(idx, emb_table)
    return out.reshape(L, B, DIM)
